# baseline (speedup 1.0000x reference)
import functools

import jax
import jax.numpy as jnp
from jax import lax
from jax.experimental import pallas as pl
from jax.experimental.pallas import tpu as pltpu
from jax.experimental.pallas import tpu_sc as plsc

_GRID = 7
_NCELL = _GRID * _GRID
_NCH = 30
_NCLS = 20
_STRIDE = 64.0
_CONF_T = 0.5
_IOU_T = 0.5
_L = 16
_NCHUNK = 4

_GDN = lax.GatherDimensionNumbers(
    offset_dims=(), collapsed_slice_dims=(0,), start_index_map=(0,))


def _sig(v):
    return 1.0 / (1.0 + jnp.exp(-v))


def _perm(v, idx):
    return lax.gather(v, idx.reshape(_L, 1), _GDN, (1,),
                      mode=lax.GatherScatterMode.PROMISE_IN_BOUNDS)


def _xlane(v, op, lane):
    for s in (1, 2, 4, 8):
        v = op(v, _perm(v, lane ^ s))
    return v


def _yolo_body(x_hbm, out_hbm, xv, x1r, y1r, x2r, y2r, arear, cxr, cyr, wr,
               hr, confr, clsr, keepr, outv):
    @pl.when((lax.axis_index("c") == 0) & (lax.axis_index("s") == 0))
    def _():
        pltpu.sync_copy(x_hbm, xv)

        lane = lax.iota(jnp.int32, _L)
        zeros = jnp.zeros((_L,), jnp.float32)
        zeroi = jnp.zeros((_L,), jnp.int32)
        ones = jnp.ones((_L,), jnp.float32)
        lane0 = lane == 0

        def gather_ch(idxv, c):
            return plsc.load_gather(
                xv, [zeroi, idxv * _NCH + c], mask=idxv < _NCELL)

        def cls_step(c, carry):
            mvs, cls_ = carry
            cf = c.astype(jnp.float32)
            out_mv, out_cl = [], []
            for j in range(_NCHUNK):
                v = _sig(gather_ch(lane + _L * j, c))
                out_cl.append(jnp.where(v > mvs[j], cf, cls_[j]))
                out_mv.append(jnp.maximum(mvs[j], v))
            return (tuple(out_mv), tuple(out_cl))

        mv0 = tuple(_sig(gather_ch(lane + _L * j, 0)) for j in range(_NCHUNK))
        _, clsv = lax.fori_loop(1, _NCLS, cls_step, (mv0, (zeros,) * _NCHUNK))
        for j in range(_NCHUNK):
            clsr[pl.ds(_L * j, _L)] = clsv[j]

        def decode_step(j, carry):
            sl = pl.ds(j * _L, _L)
            idxv = lane + j * _L
            ch = lambda c: gather_ch(idxv, c)
            c0 = _sig(ch(_NCLS))
            c1 = _sig(ch(_NCLS + 5))
            sel = c0 >= c1
            conf = jnp.where(sel, c0, c1)
            bx = _sig(jnp.where(sel, ch(_NCLS + 1), ch(_NCLS + 6)))
            by = _sig(jnp.where(sel, ch(_NCLS + 2), ch(_NCLS + 7)))
            bw = _sig(jnp.where(sel, ch(_NCLS + 3), ch(_NCLS + 8)))
            bh = _sig(jnp.where(sel, ch(_NCLS + 4), ch(_NCLS + 9)))
            gx = (idxv % _GRID).astype(jnp.float32)
            gy = (idxv // _GRID).astype(jnp.float32)
            cx = (bx + gx) * _STRIDE
            cy = (by + gy) * _STRIDE
            w = (bw * float(_GRID)) * _STRIDE
            h = (bh * float(_GRID)) * _STRIDE
            x1 = cx - w / 2.0
            y1 = cy - h / 2.0
            x2 = cx + w / 2.0
            y2 = cy + h / 2.0
            x1r[sl] = x1
            y1r[sl] = y1
            x2r[sl] = x2
            y2r[sl] = y2
            arear[sl] = (x2 - x1) * (y2 - y1)
            cxr[sl] = cx
            cyr[sl] = cy
            wr[sl] = w
            hr[sl] = h
            confr[sl] = conf
            keepr[sl] = zeros
            return carry

        lax.fori_loop(0, _NCHUNK, decode_step, jnp.int32(0))

        x1v, y1v, x2v, y2v, areav, crv = [], [], [], [], [], []
        for j in range(_NCHUNK):
            sl = pl.ds(_L * j, _L)
            x1v.append(x1r[sl])
            y1v.append(y1r[sl])
            x2v.append(x2r[sl])
            y2v.append(y2r[sl])
            areav.append(arear[sl])
            crv.append(jnp.where(lane + _L * j < _NCELL, confr[sl], 0.0))

        def _maxv(cr):
            mxv = jnp.maximum(jnp.maximum(cr[0], cr[1]),
                              jnp.maximum(cr[2], cr[3]))
            return _xlane(mxv, jnp.maximum, lane)

        def nms_cond(carry):
            return carry[0][0] > _CONF_T

        def nms_step(carry):
            mxv, cr = carry[0], carry[1:]
            candv = jnp.full((_L,), _NCELL - 1, jnp.int32)
            for j in range(_NCHUNK):
                hit = jnp.where(cr[j] == mxv, lane + _L * j, _NCELL - 1)
                candv = jnp.minimum(candv, hit)
            candv = _xlane(candv, jnp.minimum, lane)
            plsc.store_scatter(keepr, [candv], ones, mask=lane0)
            bx1 = plsc.load_gather(x1r, [candv])
            by1 = plsc.load_gather(y1r, [candv])
            bx2 = plsc.load_gather(x2r, [candv])
            by2 = plsc.load_gather(y2r, [candv])
            ba = plsc.load_gather(arear, [candv])
            out = []
            for j in range(_NCHUNK):
                ix1 = jnp.maximum(x1v[j], bx1)
                iy1 = jnp.maximum(y1v[j], by1)
                ix2 = jnp.minimum(x2v[j], bx2)
                iy2 = jnp.minimum(y2v[j], by2)
                inter = jnp.maximum(ix2 - ix1, 0.0) * jnp.maximum(iy2 - iy1, 0.0)
                iou = inter / (areav[j] + ba - inter + 1e-9)
                gone = (iou > _IOU_T) | (lane + _L * j == candv)
                out.append(jnp.where(gone, 0.0, cr[j]))
            return (_maxv(out), *out)

        lax.while_loop(nms_cond, nms_step, (_maxv(crv), *crv))

        def out_step(j, carry):
            sl = pl.ds(j * _L, _L)
            idxv = lane + j * _L
            inb = idxv < _NCELL
            kp = keepr[sl]
            for col, ref in enumerate((cxr, cyr, wr, hr, confr, clsr)):
                colv = jnp.full((_L,), col, jnp.int32)
                plsc.store_scatter(outv, [idxv, colv], ref[sl] * kp, mask=inb)
            return carry

        lax.fori_loop(0, _NCHUNK, out_step, jnp.int32(0))

        pltpu.sync_copy(outv, out_hbm)


_vmem64 = lambda: pltpu.VMEM((_NCHUNK * _L,), jnp.float32)

_yolo_sc = functools.partial(
    pl.kernel,
    out_type=jax.ShapeDtypeStruct((_NCELL, 6), jnp.float32),
    mesh=plsc.VectorSubcoreMesh(core_axis_name="c", subcore_axis_name="s",
                                num_cores=1, num_subcores=1),
    compiler_params=pltpu.CompilerParams(needs_layout_passes=False,
                                         skip_device_barrier=True),
    scratch_types=[
        pltpu.VMEM((1, _NCELL * _NCH), jnp.float32),
        _vmem64(), _vmem64(), _vmem64(), _vmem64(), _vmem64(),
        _vmem64(), _vmem64(), _vmem64(), _vmem64(),
        _vmem64(), _vmem64(), _vmem64(),
        pltpu.VMEM((_NCELL, 6), jnp.float32),
    ],
)(_yolo_body)


@jax.jit
def kernel(x):
    return _yolo_sc(x)


# speedup vs baseline: 1.0098x; 1.0098x over previous
import functools

import jax
import jax.numpy as jnp
from jax import lax
from jax.experimental import pallas as pl
from jax.experimental.pallas import tpu as pltpu
from jax.experimental.pallas import tpu_sc as plsc

_GRID = 7
_NCELL = _GRID * _GRID
_NCH = 30
_NCLS = 20
_STRIDE = 64.0
_CONF_T = 0.5
_IOU_T = 0.5
_L = 16
_NCHUNK = 4

_GDN = lax.GatherDimensionNumbers(
    offset_dims=(), collapsed_slice_dims=(0,), start_index_map=(0,))


def _sig(v):
    return 1.0 / (1.0 + jnp.exp(-v))


def _perm(v, idx):
    return lax.gather(v, idx.reshape(_L, 1), _GDN, (1,),
                      mode=lax.GatherScatterMode.PROMISE_IN_BOUNDS)


def _xlane(v, op, lane):
    for s in (1, 2, 4, 8):
        v = op(v, _perm(v, lane ^ s))
    return v


def _yolo_body(x_hbm, out_hbm, xv, x1r, y1r, x2r, y2r, arear, cxr, cyr, wr,
               hr, confr, clsr, keepr, outv):
    @pl.when((lax.axis_index("c") == 0) & (lax.axis_index("s") == 0))
    def _():
        pltpu.sync_copy(x_hbm, xv)

        lane = lax.iota(jnp.int32, _L)
        zeros = jnp.zeros((_L,), jnp.float32)
        zeroi = jnp.zeros((_L,), jnp.int32)
        ones = jnp.ones((_L,), jnp.float32)
        lane0 = lane == 0

        def gather_ch(idxv, c):
            return plsc.load_gather(
                xv, [zeroi, idxv * _NCH + c], mask=idxv < _NCELL)

        def cls_step(c, carry):
            mvs, cls_ = carry
            cf = c.astype(jnp.float32)
            out_mv, out_cl = [], []
            for j in range(_NCHUNK):
                v = _sig(gather_ch(lane + _L * j, c))
                out_cl.append(jnp.where(v > mvs[j], cf, cls_[j]))
                out_mv.append(jnp.maximum(mvs[j], v))
            return (tuple(out_mv), tuple(out_cl))

        mv0 = tuple(_sig(gather_ch(lane + _L * j, 0)) for j in range(_NCHUNK))
        _, clsv = lax.fori_loop(1, _NCLS, cls_step, (mv0, (zeros,) * _NCHUNK))
        for j in range(_NCHUNK):
            clsr[pl.ds(_L * j, _L)] = clsv[j]

        def decode_step(j, carry):
            sl = pl.ds(j * _L, _L)
            idxv = lane + j * _L
            ch = lambda c: gather_ch(idxv, c)
            c0 = _sig(ch(_NCLS))
            c1 = _sig(ch(_NCLS + 5))
            sel = c0 >= c1
            conf = jnp.where(sel, c0, c1)
            bx = _sig(jnp.where(sel, ch(_NCLS + 1), ch(_NCLS + 6)))
            by = _sig(jnp.where(sel, ch(_NCLS + 2), ch(_NCLS + 7)))
            bw = _sig(jnp.where(sel, ch(_NCLS + 3), ch(_NCLS + 8)))
            bh = _sig(jnp.where(sel, ch(_NCLS + 4), ch(_NCLS + 9)))
            gx = (idxv % _GRID).astype(jnp.float32)
            gy = (idxv // _GRID).astype(jnp.float32)
            cx = (bx + gx) * _STRIDE
            cy = (by + gy) * _STRIDE
            w = (bw * float(_GRID)) * _STRIDE
            h = (bh * float(_GRID)) * _STRIDE
            x1 = cx - w / 2.0
            y1 = cy - h / 2.0
            x2 = cx + w / 2.0
            y2 = cy + h / 2.0
            x1r[sl] = x1
            y1r[sl] = y1
            x2r[sl] = x2
            y2r[sl] = y2
            arear[sl] = (x2 - x1) * (y2 - y1)
            cxr[sl] = cx
            cyr[sl] = cy
            wr[sl] = w
            hr[sl] = h
            confr[sl] = conf
            keepr[sl] = zeros
            return carry

        lax.fori_loop(0, _NCHUNK, decode_step, jnp.int32(0))

        half_bits = jnp.int32(0x3F000000)
        x1v, y1v, x2v, y2v, areav, keyv = [], [], [], [], [], []
        for j in range(_NCHUNK):
            sl = pl.ds(_L * j, _L)
            idxv = lane + _L * j
            x1v.append(x1r[sl])
            y1v.append(y1r[sl])
            x2v.append(x2r[sl])
            y2v.append(y2r[sl])
            areav.append(arear[sl])
            bits = jnp.maximum(
                lax.bitcast_convert_type(confr[sl], jnp.int32) - half_bits, -1)
            key = (bits << 6) | (63 - idxv)
            keyv.append(jnp.where(idxv < _NCELL, key, -1))

        def _maxk(ks):
            mk = jnp.maximum(jnp.maximum(ks[0], ks[1]),
                             jnp.maximum(ks[2], ks[3]))
            return _xlane(mk, jnp.maximum, lane)

        def nms_cond(carry):
            return carry[0][0] >= 64

        def nms_step(carry):
            mxk, ks = carry[0], carry[1:]
            candv = 63 - (mxk & 63)
            plsc.store_scatter(keepr, [candv], ones, mask=lane0)
            bx1 = plsc.load_gather(x1r, [candv])
            by1 = plsc.load_gather(y1r, [candv])
            bx2 = plsc.load_gather(x2r, [candv])
            by2 = plsc.load_gather(y2r, [candv])
            ba = plsc.load_gather(arear, [candv])
            out = []
            for j in range(_NCHUNK):
                ix1 = jnp.maximum(x1v[j], bx1)
                iy1 = jnp.maximum(y1v[j], by1)
                ix2 = jnp.minimum(x2v[j], bx2)
                iy2 = jnp.minimum(y2v[j], by2)
                inter = jnp.maximum(ix2 - ix1, 0.0) * jnp.maximum(iy2 - iy1, 0.0)
                iou = inter / (areav[j] + ba - inter + 1e-9)
                gone = (iou > _IOU_T) | (ks[j] == mxk)
                out.append(jnp.where(gone, -1, ks[j]))
            return (_maxk(out), *out)

        lax.while_loop(nms_cond, nms_step, (_maxk(keyv), *keyv))

        def out_step(j, carry):
            sl = pl.ds(j * _L, _L)
            idxv = lane + j * _L
            inb = idxv < _NCELL
            kp = keepr[sl]
            for col, ref in enumerate((cxr, cyr, wr, hr, confr, clsr)):
                colv = jnp.full((_L,), col, jnp.int32)
                plsc.store_scatter(outv, [idxv, colv], ref[sl] * kp, mask=inb)
            return carry

        lax.fori_loop(0, _NCHUNK, out_step, jnp.int32(0))

        pltpu.sync_copy(outv, out_hbm)


_vmem64 = lambda: pltpu.VMEM((_NCHUNK * _L,), jnp.float32)

_yolo_sc = functools.partial(
    pl.kernel,
    out_type=jax.ShapeDtypeStruct((_NCELL, 6), jnp.float32),
    mesh=plsc.VectorSubcoreMesh(core_axis_name="c", subcore_axis_name="s",
                                num_cores=1, num_subcores=1),
    compiler_params=pltpu.CompilerParams(needs_layout_passes=False,
                                         skip_device_barrier=True),
    scratch_types=[
        pltpu.VMEM((1, _NCELL * _NCH), jnp.float32),
        _vmem64(), _vmem64(), _vmem64(), _vmem64(), _vmem64(),
        _vmem64(), _vmem64(), _vmem64(), _vmem64(),
        _vmem64(), _vmem64(), _vmem64(),
        pltpu.VMEM((_NCELL, 6), jnp.float32),
    ],
)(_yolo_body)


@jax.jit
def kernel(x):
    return _yolo_sc(x)


# speedup vs baseline: 1.0131x; 1.0033x over previous
import functools

import jax
import jax.numpy as jnp
from jax import lax
from jax.experimental import pallas as pl
from jax.experimental.pallas import tpu as pltpu
from jax.experimental.pallas import tpu_sc as plsc

_GRID = 7
_NCELL = _GRID * _GRID
_NCH = 30
_NCLS = 20
_STRIDE = 64.0
_IOU_T = 0.5
_L = 16
_NCHUNK = 4

_X1, _Y1, _X2, _Y2, _AREA, _CX, _CY, _W, _H, _CONF, _CLS, _KEEP = (
    i * _NCHUNK * _L for i in range(12))

_GDN = lax.GatherDimensionNumbers(
    offset_dims=(), collapsed_slice_dims=(0,), start_index_map=(0,))


def _sig(v):
    return 1.0 / (1.0 + jnp.exp(-v))


def _perm(v, idx):
    return lax.gather(v, idx.reshape(_L, 1), _GDN, (1,),
                      mode=lax.GatherScatterMode.PROMISE_IN_BOUNDS)


def _xlane_max(v, lane):
    for s in (1, 2, 4, 8):
        v = jnp.maximum(v, _perm(v, lane ^ s))
    return v


def _yolo_body(x_hbm, out_hbm, xv, br, outv):
    @pl.when((lax.axis_index("c") == 0) & (lax.axis_index("s") == 0))
    def _():
        pltpu.sync_copy(x_hbm, xv)

        lane = lax.iota(jnp.int32, _L)
        zeros = jnp.zeros((_L,), jnp.float32)
        zeroi = jnp.zeros((_L,), jnp.int32)
        ones = jnp.ones((_L,), jnp.float32)
        lane0 = lane == 0

        def gather_ch(idxv, c):
            return plsc.load_gather(
                xv, [zeroi, idxv * _NCH + c], mask=idxv < _NCELL)

        def cls_step(c, carry):
            mvs, cls_ = carry
            cf = c.astype(jnp.float32)
            out_mv, out_cl = [], []
            for j in range(_NCHUNK):
                v = _sig(gather_ch(lane + _L * j, c))
                out_cl.append(jnp.where(v > mvs[j], cf, cls_[j]))
                out_mv.append(jnp.maximum(mvs[j], v))
            return (tuple(out_mv), tuple(out_cl))

        mv0 = tuple(_sig(gather_ch(lane + _L * j, 0)) for j in range(_NCHUNK))
        _, clsv = lax.fori_loop(1, _NCLS, cls_step, (mv0, (zeros,) * _NCHUNK))
        for j in range(_NCHUNK):
            br[pl.ds(_CLS + _L * j, _L)] = clsv[j]

        def decode_step(j, carry):
            idxv = lane + j * _L
            ch = lambda c: gather_ch(idxv, c)
            c0 = _sig(ch(_NCLS))
            c1 = _sig(ch(_NCLS + 5))
            sel = c0 >= c1
            conf = jnp.where(sel, c0, c1)
            bx = _sig(jnp.where(sel, ch(_NCLS + 1), ch(_NCLS + 6)))
            by = _sig(jnp.where(sel, ch(_NCLS + 2), ch(_NCLS + 7)))
            bw = _sig(jnp.where(sel, ch(_NCLS + 3), ch(_NCLS + 8)))
            bh = _sig(jnp.where(sel, ch(_NCLS + 4), ch(_NCLS + 9)))
            gx = (idxv % _GRID).astype(jnp.float32)
            gy = (idxv // _GRID).astype(jnp.float32)
            cx = (bx + gx) * _STRIDE
            cy = (by + gy) * _STRIDE
            w = (bw * float(_GRID)) * _STRIDE
            h = (bh * float(_GRID)) * _STRIDE
            x1 = cx - w / 2.0
            y1 = cy - h / 2.0
            x2 = cx + w / 2.0
            y2 = cy + h / 2.0
            off = j * _L
            br[pl.ds(_X1 + off, _L)] = x1
            br[pl.ds(_Y1 + off, _L)] = y1
            br[pl.ds(_X2 + off, _L)] = x2
            br[pl.ds(_Y2 + off, _L)] = y2
            br[pl.ds(_AREA + off, _L)] = (x2 - x1) * (y2 - y1)
            br[pl.ds(_CX + off, _L)] = cx
            br[pl.ds(_CY + off, _L)] = cy
            br[pl.ds(_W + off, _L)] = w
            br[pl.ds(_H + off, _L)] = h
            br[pl.ds(_CONF + off, _L)] = conf
            br[pl.ds(_KEEP + off, _L)] = zeros
            return carry

        lax.fori_loop(0, _NCHUNK, decode_step, jnp.int32(0))

        half_bits = jnp.int32(0x3F000000)
        x1v, y1v, x2v, y2v, areav, keyv = [], [], [], [], [], []
        for j in range(_NCHUNK):
            off = j * _L
            idxv = lane + off
            x1v.append(br[pl.ds(_X1 + off, _L)])
            y1v.append(br[pl.ds(_Y1 + off, _L)])
            x2v.append(br[pl.ds(_X2 + off, _L)])
            y2v.append(br[pl.ds(_Y2 + off, _L)])
            areav.append(br[pl.ds(_AREA + off, _L)])
            conf = br[pl.ds(_CONF + off, _L)]
            bits = jnp.maximum(
                lax.bitcast_convert_type(conf, jnp.int32) - half_bits, -1)
            key = (bits << 6) | (63 - idxv)
            keyv.append(jnp.where(idxv < _NCELL, key, -1))

        def _maxk(ks):
            mk = jnp.maximum(jnp.maximum(ks[0], ks[1]),
                             jnp.maximum(ks[2], ks[3]))
            return _xlane_max(mk, lane)

        def nms_cond(carry):
            return carry[0][0] >= 64

        def nms_step(carry):
            mxk, ks = carry[0], carry[1:]
            candv = 63 - (mxk & 63)
            plsc.store_scatter(br, [candv + _KEEP], ones, mask=lane0)
            bx1 = plsc.load_gather(br, [candv])
            by1 = plsc.load_gather(br, [candv + _Y1])
            bx2 = plsc.load_gather(br, [candv + _X2])
            by2 = plsc.load_gather(br, [candv + _Y2])
            ba = plsc.load_gather(br, [candv + _AREA])
            out = []
            for j in range(_NCHUNK):
                ix1 = jnp.maximum(x1v[j], bx1)
                iy1 = jnp.maximum(y1v[j], by1)
                ix2 = jnp.minimum(x2v[j], bx2)
                iy2 = jnp.minimum(y2v[j], by2)
                inter = jnp.maximum(ix2 - ix1, 0.0) * jnp.maximum(iy2 - iy1, 0.0)
                iou = inter / (areav[j] + ba - inter + 1e-9)
                gone = (iou > _IOU_T) | (ks[j] == mxk)
                out.append(jnp.where(gone, -1, ks[j]))
            return (_maxk(out), *out)

        lax.while_loop(nms_cond, nms_step, (_maxk(keyv), *keyv))

        def out_step(j, carry):
            off = j * _L
            idxv = lane + off
            inb = idxv < _NCELL
            kp = br[pl.ds(_KEEP + off, _L)]
            for col, fo in enumerate((_CX, _CY, _W, _H, _CONF, _CLS)):
                colv = jnp.full((_L,), col, jnp.int32)
                plsc.store_scatter(outv, [idxv, colv],
                                   br[pl.ds(fo + off, _L)] * kp, mask=inb)
            return carry

        lax.fori_loop(0, _NCHUNK, out_step, jnp.int32(0))

        pltpu.sync_copy(outv, out_hbm)


_yolo_sc = functools.partial(
    pl.kernel,
    out_type=jax.ShapeDtypeStruct((_NCELL, 6), jnp.float32),
    mesh=plsc.VectorSubcoreMesh(core_axis_name="c", subcore_axis_name="s",
                                num_cores=1, num_subcores=1),
    compiler_params=pltpu.CompilerParams(needs_layout_passes=False,
                                         skip_device_barrier=True),
    scratch_types=[
        pltpu.VMEM((1, _NCELL * _NCH), jnp.float32),
        pltpu.VMEM((12 * _NCHUNK * _L,), jnp.float32),
        pltpu.VMEM((_NCELL, 6), jnp.float32),
    ],
)(_yolo_body)


@jax.jit
def kernel(x):
    return _yolo_sc(x)


# speedup vs baseline: 1.0201x; 1.0069x over previous
import functools

import jax
import jax.numpy as jnp
from jax import lax
from jax.experimental import pallas as pl
from jax.experimental.pallas import tpu as pltpu
from jax.experimental.pallas import tpu_sc as plsc

_GRID = 7
_NCELL = _GRID * _GRID
_NCH = 30
_NCLS = 20
_STRIDE = 64.0
_IOU_T = 0.5
_L = 16
_NCHUNK = 4

_X1, _Y1, _X2, _Y2, _AREA, _CX, _CY, _W, _H, _CONF, _CLS, _KEEP = (
    i * _NCHUNK * _L for i in range(12))

_GDN = lax.GatherDimensionNumbers(
    offset_dims=(), collapsed_slice_dims=(0,), start_index_map=(0,))


def _sig(v):
    return 1.0 / (1.0 + jnp.exp(-v))


def _perm(v, idx):
    return lax.gather(v, idx.reshape(_L, 1), _GDN, (1,),
                      mode=lax.GatherScatterMode.PROMISE_IN_BOUNDS)


def _xlane_max(v, lane):
    for s in (1, 2, 4, 8):
        v = jnp.maximum(v, _perm(v, lane ^ s))
    return v


def _yolo_body(x_hbm, out_hbm, xv, br, outv):
    @pl.when((lax.axis_index("c") == 0) & (lax.axis_index("s") == 0))
    def _():
        pltpu.sync_copy(x_hbm, xv)

        lane = lax.iota(jnp.int32, _L)
        zeros = jnp.zeros((_L,), jnp.float32)
        zeroi = jnp.zeros((_L,), jnp.int32)
        ones = jnp.ones((_L,), jnp.float32)
        lane0 = lane == 0

        def gather_ch(idxv, c):
            return plsc.load_gather(
                xv, [zeroi, idxv * _NCH + c], mask=idxv < _NCELL)

        def cls_step(c, carry):
            mvs, cls_ = carry
            cf = c.astype(jnp.float32)
            out_mv, out_cl = [], []
            for j in range(_NCHUNK):
                v = _sig(gather_ch(lane + _L * j, c))
                out_cl.append(jnp.where(v > mvs[j], cf, cls_[j]))
                out_mv.append(jnp.maximum(mvs[j], v))
            return (tuple(out_mv), tuple(out_cl))

        mv0 = tuple(_sig(gather_ch(lane + _L * j, 0)) for j in range(_NCHUNK))
        _, clsv = lax.fori_loop(1, _NCLS, cls_step, (mv0, (zeros,) * _NCHUNK))
        for j in range(_NCHUNK):
            br[pl.ds(_CLS + _L * j, _L)] = clsv[j]

        def decode_step(j, carry):
            idxv = lane + j * _L
            ch = lambda c: gather_ch(idxv, c)
            c0 = _sig(ch(_NCLS))
            c1 = _sig(ch(_NCLS + 5))
            sel = c0 >= c1
            conf = jnp.where(sel, c0, c1)
            bx = _sig(jnp.where(sel, ch(_NCLS + 1), ch(_NCLS + 6)))
            by = _sig(jnp.where(sel, ch(_NCLS + 2), ch(_NCLS + 7)))
            bw = _sig(jnp.where(sel, ch(_NCLS + 3), ch(_NCLS + 8)))
            bh = _sig(jnp.where(sel, ch(_NCLS + 4), ch(_NCLS + 9)))
            gx = (idxv % _GRID).astype(jnp.float32)
            gy = (idxv // _GRID).astype(jnp.float32)
            cx = (bx + gx) * _STRIDE
            cy = (by + gy) * _STRIDE
            w = (bw * float(_GRID)) * _STRIDE
            h = (bh * float(_GRID)) * _STRIDE
            x1 = cx - w / 2.0
            y1 = cy - h / 2.0
            x2 = cx + w / 2.0
            y2 = cy + h / 2.0
            off = j * _L
            br[pl.ds(_X1 + off, _L)] = x1
            br[pl.ds(_Y1 + off, _L)] = y1
            br[pl.ds(_X2 + off, _L)] = x2
            br[pl.ds(_Y2 + off, _L)] = y2
            br[pl.ds(_AREA + off, _L)] = (x2 - x1) * (y2 - y1)
            br[pl.ds(_CX + off, _L)] = cx
            br[pl.ds(_CY + off, _L)] = cy
            br[pl.ds(_W + off, _L)] = w
            br[pl.ds(_H + off, _L)] = h
            br[pl.ds(_CONF + off, _L)] = conf
            br[pl.ds(_KEEP + off, _L)] = zeros
            return carry

        lax.fori_loop(0, _NCHUNK, decode_step, jnp.int32(0))

        half_bits = jnp.int32(0x3F000000)
        x1v, y1v, x2v, y2v, areav, keyv = [], [], [], [], [], []
        for j in range(_NCHUNK):
            off = j * _L
            idxv = lane + off
            x1v.append(br[pl.ds(_X1 + off, _L)])
            y1v.append(br[pl.ds(_Y1 + off, _L)])
            x2v.append(br[pl.ds(_X2 + off, _L)])
            y2v.append(br[pl.ds(_Y2 + off, _L)])
            areav.append(br[pl.ds(_AREA + off, _L)])
            conf = br[pl.ds(_CONF + off, _L)]
            bits = jnp.maximum(
                lax.bitcast_convert_type(conf, jnp.int32) - half_bits, -1)
            key = (bits << 6) | (63 - idxv)
            keyv.append(jnp.where(idxv < _NCELL, key, -1))

        def _maxk(ks):
            mk = jnp.maximum(jnp.maximum(ks[0], ks[1]),
                             jnp.maximum(ks[2], ks[3]))
            return _xlane_max(mk, lane)

        def nms_cond(carry):
            return carry[0][0] >= 64

        def nms_step(carry):
            mxk, ks = carry[0], carry[1:]
            candv = 63 - (mxk & 63)
            plsc.store_scatter(br, [candv + _KEEP], ones, mask=lane0)
            bx1 = plsc.load_gather(br, [candv])
            by1 = plsc.load_gather(br, [candv + _Y1])
            bx2 = plsc.load_gather(br, [candv + _X2])
            by2 = plsc.load_gather(br, [candv + _Y2])
            ba = plsc.load_gather(br, [candv + _AREA])
            out = []
            for j in range(_NCHUNK):
                ix1 = jnp.maximum(x1v[j], bx1)
                iy1 = jnp.maximum(y1v[j], by1)
                ix2 = jnp.minimum(x2v[j], bx2)
                iy2 = jnp.minimum(y2v[j], by2)
                inter = jnp.maximum(ix2 - ix1, 0.0) * jnp.maximum(iy2 - iy1, 0.0)
                iou = inter / (areav[j] + ba - inter + 1e-9)
                gone = (iou > _IOU_T) | (ks[j] == mxk)
                out.append(jnp.where(gone, -1, ks[j]))
            return (_maxk(out), *out)

        lax.while_loop(nms_cond, nms_step, (_maxk(keyv), *keyv))

        def out_step(j, carry):
            off = j * _L
            idxv = lane + off
            inb = idxv < _NCELL
            kp = br[pl.ds(_KEEP + off, _L)]
            for col, fo in enumerate((_CX, _CY, _W, _H, _CONF, _CLS)):
                colv = jnp.full((_L,), col, jnp.int32)
                plsc.store_scatter(outv, [idxv, colv],
                                   br[pl.ds(fo + off, _L)] * kp, mask=inb)
            return carry

        lax.fori_loop(0, _NCHUNK, out_step, jnp.int32(0))

        pltpu.sync_copy(outv, out_hbm)


_yolo_sc = functools.partial(
    pl.kernel,
    out_type=jax.ShapeDtypeStruct((_NCELL, 6), jnp.float32),
    mesh=plsc.VectorSubcoreMesh(core_axis_name="c", subcore_axis_name="s",
                                num_cores=1, num_subcores=1),
    compiler_params=pltpu.CompilerParams(needs_layout_passes=False,
                                         skip_device_barrier=True,
                                         disable_bounds_checks=True),
    scratch_types=[
        pltpu.VMEM((1, _NCELL * _NCH), jnp.float32),
        pltpu.VMEM((12 * _NCHUNK * _L,), jnp.float32),
        pltpu.VMEM((_NCELL, 6), jnp.float32),
    ],
)(_yolo_body)


@jax.jit
def kernel(x):
    return _yolo_sc(x)
